# baseline (speedup 1.0000x reference)
import functools

import jax
import jax.numpy as jnp
from jax import lax
from jax.experimental import pallas as pl
from jax.experimental.pallas import tpu as pltpu
from jax.experimental.pallas import tpu_sc as plsc

ZMAX, YMAX, XMAX = 40, 400, 352
GRID = ZMAX * YMAX * XMAX
VX, VY, VZ = 0.2, 0.2, 0.1
XMIN, YMIN, ZMIN = 0.0, -40.0, -2.0
M = 16384
N = 16384
C = 64
NS = 27
ROWW = 72
PAIRS = N * NS

NCORE, NSUB = 2, 16
NW = NCORE * NSUB

MEMSET_PER_W = GRID // NSUB
MEMSET_BUF = 16384
VOX_PER_W2 = M // NSUB

PTS_PER_W = N // NW
IDX_PER_W = PTS_PER_W * NS
CHUNK = 128
NCHUNK = IDX_PER_W // CHUNK

_OFFS = [(a, b, c) for a in (-1, 0, 1) for b in (-1, 0, 1) for c in (-1, 0, 1)]

_f32 = jnp.float32
_i32 = jnp.int32


def _k1_body(vi_ref, vf_ref, raw_ref):
    vi = vi_ref[...]
    b = vi.shape[0]
    xf = (vi[:, 3:4].astype(_f32) + 0.5) * VX + XMIN
    yf = (vi[:, 2:3].astype(_f32) + 0.5) * VY + YMIN
    zf = (vi[:, 1:2].astype(_f32) + 0.5) * VZ + ZMIN
    qf = xf * xf + yf * yf + zf * zf
    raw_ref[...] = jnp.concatenate(
        [xf, yf, zf, qf, vf_ref[...],
         jnp.zeros((b, ROWW - 4 - C), _f32)], axis=1)


def _build_raw(v_indices, v_features):
    blk = 1024
    return pl.pallas_call(
        _k1_body,
        grid=(M // blk,),
        in_specs=[pl.BlockSpec((blk, 4), lambda i: (i, 0)),
                  pl.BlockSpec((blk, C), lambda i: (i, 0))],
        out_specs=pl.BlockSpec((blk, ROWW), lambda i: (i, 0)),
        out_shape=jax.ShapeDtypeStruct((M, ROWW), _f32),
    )(v_indices, v_features)


def _k2_body(zi_hbm, yi_hbm, xi_hbm, grid_hbm,
             negbuf, zv, yv, xv, linb, valb, sem):
    cid = lax.axis_index("c")
    sid = lax.axis_index("s")

    @pl.when(cid == 0)
    def _memset():
        def fill(i, carry):
            negbuf[pl.ds(i * 16, 16)] = jnp.full((16,), -1, _i32)
            return carry
        lax.fori_loop(0, MEMSET_BUF // 16, fill, 0)
        base = sid * MEMSET_PER_W
        nfull = MEMSET_PER_W // MEMSET_BUF
        tail = MEMSET_PER_W - nfull * MEMSET_BUF

        def fire(i, carry):
            pltpu.async_copy(negbuf, grid_hbm.at[pl.ds(base + i * MEMSET_BUF,
                                                       MEMSET_BUF)], sem)
            return carry
        lax.fori_loop(0, nfull, fire, 0)
        if tail:
            pltpu.async_copy(negbuf.at[pl.ds(0, tail)],
                             grid_hbm.at[pl.ds(base + nfull * MEMSET_BUF, tail)],
                             sem)

        def drain(i, carry):
            pltpu.make_async_copy(
                negbuf, grid_hbm.at[pl.ds(base + i * MEMSET_BUF, MEMSET_BUF)],
                sem).wait()
            return carry
        lax.fori_loop(0, nfull, drain, 0)
        if tail:
            pltpu.make_async_copy(
                negbuf.at[pl.ds(0, tail)],
                grid_hbm.at[pl.ds(base + nfull * MEMSET_BUF, tail)], sem).wait()

    plsc.subcore_barrier()

    @pl.when(cid == 0)
    def _scatter():
        vbase = sid * VOX_PER_W2
        pltpu.sync_copy(zi_hbm.at[pl.ds(vbase, VOX_PER_W2)], zv)
        pltpu.sync_copy(yi_hbm.at[pl.ds(vbase, VOX_PER_W2)], yv)
        pltpu.sync_copy(xi_hbm.at[pl.ds(vbase, VOX_PER_W2)], xv)
        iot = lax.iota(_i32, 16)
        for g in range(VOX_PER_W2 // 16):
            z16 = zv[pl.ds(g * 16, 16)]
            y16 = yv[pl.ds(g * 16, 16)]
            x16 = xv[pl.ds(g * 16, 16)]
            lin = z16 * (YMAX * XMAX) + y16 * XMAX + x16
            val = vbase + g * 16 + iot
            j, col = divmod(g * 16, CHUNK)
            linb[j, pl.ds(col, 16)] = lin
            valb[j, pl.ds(col, 16)] = val
        nscat = VOX_PER_W2 // CHUNK
        for j in range(nscat):
            pltpu.async_copy(valb.at[j], grid_hbm.at[linb.at[j]], sem)
        for j in range(nscat):
            pltpu.make_async_copy(valb.at[j], grid_hbm.at[linb.at[j]],
                                  sem).wait()


def _build_grid(zi, yi, xi):
    mesh = plsc.VectorSubcoreMesh(core_axis_name="c", subcore_axis_name="s",
                                  num_cores=NCORE, num_subcores=NSUB)
    return pl.kernel(
        _k2_body,
        out_type=jax.ShapeDtypeStruct((GRID,), _i32),
        mesh=mesh,
        scratch_types=[
            pltpu.VMEM((MEMSET_BUF,), _i32),
            pltpu.VMEM((VOX_PER_W2,), _i32),
            pltpu.VMEM((VOX_PER_W2,), _i32),
            pltpu.VMEM((VOX_PER_W2,), _i32),
            pltpu.VMEM((VOX_PER_W2 // CHUNK, CHUNK), _i32),
            pltpu.VMEM((VOX_PER_W2 // CHUNK, CHUNK), _i32),
            pltpu.SemaphoreType.DMA,
        ],
    )(zi, yi, xi)


NBUF = 6


def _k3_body(px_hbm, py_hbm, pz_hbm, grid_hbm, raw_hbm, grp_hbm, msk_hbm,
             pxv, pyv, pzv, linbuf, auxbuf, gbuf, *bufs_sems):
    rowbufs = bufs_sems[:NBUF]
    sem_g = bufs_sems[NBUF]
    gsems = bufs_sems[NBUF + 1:2 * NBUF + 1]
    wsems = bufs_sems[2 * NBUF + 1:]
    cid = lax.axis_index("c")
    sid = lax.axis_index("s")
    w = cid * NSUB + sid
    nbase = w * PTS_PER_W
    pbase = nbase * NS

    pltpu.sync_copy(px_hbm.at[pl.ds(nbase, PTS_PER_W)], pxv)
    pltpu.sync_copy(py_hbm.at[pl.ds(nbase, PTS_PER_W)], pyv)
    pltpu.sync_copy(pz_hbm.at[pl.ds(nbase, PTS_PER_W)], pzv)

    def grp(g, carry):
        x16 = pxv[pl.ds(g * 16, 16)]
        y16 = pyv[pl.ds(g * 16, 16)]
        z16 = pzv[pl.ds(g * 16, 16)]
        pxi = ((x16 - XMIN) / VX).astype(_i32)
        pyi = ((y16 - YMIN) / VY).astype(_i32)
        pzi = ((z16 - ZMIN) / VZ).astype(_i32)
        pxi = jnp.minimum(jnp.maximum(pxi, 0), XMAX - 1)
        pyi = jnp.minimum(jnp.maximum(pyi, 0), YMAX - 1)
        pzi = jnp.minimum(jnp.maximum(pzi, 0), ZMAX - 1)
        for k, (dz, dy, dx) in enumerate(_OFFS):
            nz = pzi + dz
            ny = pyi + dy
            nx = pxi + dx
            inb = ((nz >= 0) & (nz < ZMAX) & (ny >= 0) & (ny < YMAX)
                   & (nx >= 0) & (nx < XMAX))
            lin = nz * (YMAX * XMAX) + ny * XMAX + nx
            lin = jnp.where(inb, lin, -1)
            linbuf[pl.ds(k * PTS_PER_W + g * 16, 16)] = lin
        return carry
    lax.fori_loop(0, PTS_PER_W // 16, grp, 0)

    iot = lax.iota(_i32, 16)

    def clampf(i, carry):
        l16 = linbuf[pl.ds(i * 16, 16)]
        pad = (i * 16 + iot) * 13
        auxbuf[pl.ds(i * 16, 16)] = jnp.where(l16 >= 0, l16, pad)
        return carry
    lax.fori_loop(0, IDX_PER_W // 16, clampf, 0)

    def fire_g(c, carry):
        pltpu.async_copy(grid_hbm.at[auxbuf.at[pl.ds(c * CHUNK, CHUNK)]],
                         gbuf.at[pl.ds(c * CHUNK, CHUNK)], sem_g)
        return carry
    lax.fori_loop(0, NCHUNK, fire_g, 0)

    def drain_g(c, carry):
        pltpu.make_async_copy(grid_hbm.at[auxbuf.at[pl.ds(c * CHUNK, CHUNK)]],
                              gbuf.at[pl.ds(c * CHUNK, CHUNK)], sem_g).wait()
        return carry
    lax.fori_loop(0, NCHUNK, drain_g, 0)

    def fpass(i, carry):
        g16 = gbuf[pl.ds(i * 16, 16)]
        l16 = linbuf[pl.ds(i * 16, 16)]
        valid = (g16 >= 0) & (l16 >= 0)
        pad = (i * 16 + iot) & (M - 1)
        auxbuf[pl.ds(i * 16, 16)] = jnp.where(valid, g16, pad)
        gbuf[pl.ds(i * 16, 16)] = jnp.where(valid, 1, 0)
        return carry
    lax.fori_loop(0, IDX_PER_W // 16, fpass, 0)
    pltpu.sync_copy(gbuf, msk_hbm.at[pl.ds(w * IDX_PER_W, IDX_PER_W)])

    def ring(o, carry):
        for b in range(NBUF):
            c = o * NBUF + b

            @pl.when(o > 0)
            def _wait_wb(b=b, c=c):
                pltpu.make_async_copy(
                    rowbufs[b],
                    grp_hbm.at[pl.ds(pbase + (c - NBUF) * CHUNK, CHUNK)],
                    wsems[b]).wait()
            pltpu.async_copy(raw_hbm.at[auxbuf.at[pl.ds(c * CHUNK, CHUNK)]],
                             rowbufs[b], gsems[b])
        for b in range(NBUF):
            c = o * NBUF + b
            pltpu.make_async_copy(
                raw_hbm.at[auxbuf.at[pl.ds(c * CHUNK, CHUNK)]],
                rowbufs[b], gsems[b]).wait()
            pltpu.async_copy(rowbufs[b],
                             grp_hbm.at[pl.ds(pbase + c * CHUNK, CHUNK)],
                             wsems[b])
        return carry
    lax.fori_loop(0, NCHUNK // NBUF, ring, 0)
    for b in range(NBUF):
        c = (NCHUNK - NBUF) + b
        pltpu.make_async_copy(rowbufs[b],
                              grp_hbm.at[pl.ds(pbase + c * CHUNK, CHUNK)],
                              wsems[b]).wait()


def _gather_grouped(px, py, pz, grid, raw):
    mesh = plsc.VectorSubcoreMesh(core_axis_name="c", subcore_axis_name="s",
                                  num_cores=NCORE, num_subcores=NSUB)
    return pl.kernel(
        _k3_body,
        out_type=[jax.ShapeDtypeStruct((PAIRS, ROWW), _f32),
                  jax.ShapeDtypeStruct((PAIRS,), _i32)],
        mesh=mesh,
        compiler_params=pltpu.CompilerParams(use_tc_tiling_on_sc=False),
        scratch_types=[
            pltpu.VMEM((PTS_PER_W,), _f32),
            pltpu.VMEM((PTS_PER_W,), _f32),
            pltpu.VMEM((PTS_PER_W,), _f32),
            pltpu.VMEM((IDX_PER_W,), _i32),
            pltpu.VMEM((IDX_PER_W,), _i32),
            pltpu.VMEM((IDX_PER_W,), _i32),
        ] + [pltpu.VMEM((CHUNK, ROWW), _f32) for _ in range(NBUF)]
          + [pltpu.SemaphoreType.DMA for _ in range(2 * NBUF + 1)],
    )(px, py, pz, grid, raw)


_K4_P = 128
_K4_STEPS = N // _K4_P


_K4_J = PTS_PER_W // _K4_P


def _k4_body(grp_ref, msk_ref, r0_ref, pc_ref, wf_ref, w4_ref,
             hmax_ref, hmin_ref, sums_ref, acc_ref):
    w = pl.program_id(0)
    j = pl.program_id(1)
    raw3 = grp_ref[...][0]
    msk3d = msk_ref[...][0] > 0
    c3 = pc_ref[...][:, 1:4]
    wf = wf_ref[...]
    w4 = w4_ref[...]
    flat = raw3.reshape(NS * _K4_P, ROWW)
    base3 = jnp.dot(flat, wf, preferred_element_type=_f32).reshape(
        NS, _K4_P, C)
    r = jnp.sum(c3 * c3, axis=1, keepdims=True)
    p0 = r * w4 - jnp.dot(c3, wf[0:3, :], preferred_element_type=_f32)
    s3 = jnp.sum(raw3[:, :, 0:3] * c3[None, :, :], axis=2, keepdims=True)
    h3 = base3 + p0[None, :, :] - 2.0 * s3 * w4[None, :, :]
    r0 = r0_ref[...]
    base0 = jnp.dot(r0, wf, preferred_element_type=_f32)
    s0 = jnp.sum(r0[:, 0:3] * c3, axis=1, keepdims=True)
    h0 = base0 + p0 - 2.0 * s0 * w4
    h3 = jnp.where(msk3d, h3, h0[None, :, :])
    hmax_ref[...] = jnp.max(h3, axis=0)
    hmin_ref[...] = jnp.min(h3, axis=0)
    h2 = h3.reshape(NS * _K4_P, C)
    stat = jnp.concatenate([jnp.sum(h2, axis=0)[None, :],
                            jnp.sum(h2 * h2, axis=0)[None, :]], axis=0)

    @pl.when((w == 0) & (j == 0))
    def _init():
        acc_ref[...] = jnp.zeros((2, C), _f32)

    acc_ref[...] += stat

    @pl.when((w == NW - 1) & (j == _K4_J - 1))
    def _out():
        sums_ref[...] = acc_ref[...]


def _dense_stage(grp4, msk4, raw0, p_coords, wfull, w4row):
    return pl.pallas_call(
        _k4_body,
        grid=(NW, _K4_J),
        in_specs=[pl.BlockSpec((1, NS, _K4_P, ROWW), lambda w, j: (w, 0, j, 0)),
                  pl.BlockSpec((1, NS, _K4_P, 1), lambda w, j: (w, 0, j, 0)),
                  pl.BlockSpec((1, ROWW), lambda w, j: (0, 0)),
                  pl.BlockSpec((_K4_P, 4), lambda w, j: (w * _K4_J + j, 0)),
                  pl.BlockSpec((ROWW, C), lambda w, j: (0, 0)),
                  pl.BlockSpec((1, C), lambda w, j: (0, 0))],
        out_specs=[pl.BlockSpec((_K4_P, C), lambda w, j: (w * _K4_J + j, 0)),
                   pl.BlockSpec((_K4_P, C), lambda w, j: (w * _K4_J + j, 0)),
                   pl.BlockSpec((2, C), lambda w, j: (0, 0))],
        out_shape=[jax.ShapeDtypeStruct((N, C), _f32),
                   jax.ShapeDtypeStruct((N, C), _f32),
                   jax.ShapeDtypeStruct((2, C), _f32)],
        scratch_shapes=[pltpu.VMEM((2, C), _f32)],
    )(grp4, msk4, raw0, p_coords, wfull, w4row)


_K5_B = 1024


def _k5_body(sums_ref, hmax_ref, hmin_ref, pf_ref, g_ref, b_ref, out_ref):
    inv = _f32(1.0 / PAIRS)
    mu = sums_ref[0:1, :] * inv
    ex2 = sums_ref[1:2, :] * inv
    var = ex2 - mu * mu
    s = g_ref[...] / jnp.sqrt(var + 1e-5)
    b = b_ref[...] - mu * s
    cand = jnp.where(s >= 0, s * hmax_ref[...] + b, s * hmin_ref[...] + b)
    out_ref[...] = pf_ref[...] + jnp.maximum(cand, 0.0)


def _finalize(sums, hmax, hmin, p_features, gamma2, beta2):
    return pl.pallas_call(
        _k5_body,
        grid=(N // _K5_B,),
        in_specs=[pl.BlockSpec((2, C), lambda i: (0, 0)),
                  pl.BlockSpec((_K5_B, C), lambda i: (i, 0)),
                  pl.BlockSpec((_K5_B, C), lambda i: (i, 0)),
                  pl.BlockSpec((_K5_B, C), lambda i: (i, 0)),
                  pl.BlockSpec((1, C), lambda i: (0, 0)),
                  pl.BlockSpec((1, C), lambda i: (0, 0))],
        out_specs=pl.BlockSpec((_K5_B, C), lambda i: (i, 0)),
        out_shape=jax.ShapeDtypeStruct((N, C), _f32),
    )(sums, hmax, hmin, p_features, gamma2, beta2)


def kernel(v_features, v_indices, p_coords, p_features, W, bn_gamma, bn_beta):
    zi = v_indices[:, 1]
    yi = v_indices[:, 2]
    xi = v_indices[:, 3]
    px = p_coords[:, 1]
    py = p_coords[:, 2]
    pz = p_coords[:, 3]

    raw = _build_raw(v_indices, v_features)
    grid = _build_grid(zi, yi, xi)
    grp, msk = _gather_grouped(px, py, pz, grid, raw)
    grp4 = grp.reshape(NW, NS, PTS_PER_W, ROWW)
    msk4 = msk.reshape(NW, NS, PTS_PER_W, 1)
    raw0 = lax.slice(raw, (0, 0), (1, ROWW))
    wfull = jnp.concatenate([W.T, jnp.zeros((ROWW - 4 - C, C), _f32)], axis=0)
    w4row = W.T[3:4]
    hmax, hmin, sums = _dense_stage(grp4, msk4, raw0, p_coords, wfull, w4row)
    out = _finalize(sums, hmax, hmin, p_features,
                    bn_gamma.reshape(1, C), bn_beta.reshape(1, C))
    return out


# speedup vs baseline: 1.0547x; 1.0547x over previous
import functools

import jax
import jax.numpy as jnp
from jax import lax
from jax.experimental import pallas as pl
from jax.experimental.pallas import tpu as pltpu
from jax.experimental.pallas import tpu_sc as plsc

ZMAX, YMAX, XMAX = 40, 400, 352
GRID = ZMAX * YMAX * XMAX
VX, VY, VZ = 0.2, 0.2, 0.1
XMIN, YMIN, ZMIN = 0.0, -40.0, -2.0
M = 16384
N = 16384
C = 64
NS = 27
ROWW = 72
PAIRS = N * NS

NCORE, NSUB = 2, 16
NW = NCORE * NSUB

MEMSET_PER_W = GRID // NSUB
MEMSET_BUF = 16384
VOX_PER_W2 = M // NSUB

PTS_PER_W = N // NW
IDX_PER_W = PTS_PER_W * NS
CHUNK = 128
NCHUNK = IDX_PER_W // CHUNK

_OFFS = [(a, b, c) for a in (-1, 0, 1) for b in (-1, 0, 1) for c in (-1, 0, 1)]

_f32 = jnp.float32
_i32 = jnp.int32


def _k1_body(vi_ref, vf_ref, raw_ref):
    vi = vi_ref[...]
    b = vi.shape[0]
    xf = (vi[:, 3:4].astype(_f32) + 0.5) * VX + XMIN
    yf = (vi[:, 2:3].astype(_f32) + 0.5) * VY + YMIN
    zf = (vi[:, 1:2].astype(_f32) + 0.5) * VZ + ZMIN
    qf = xf * xf + yf * yf + zf * zf
    raw_ref[...] = jnp.concatenate(
        [xf, yf, zf, qf, vf_ref[...],
         jnp.zeros((b, ROWW - 4 - C), _f32)], axis=1)


def _build_raw(v_indices, v_features):
    blk = 1024
    return pl.pallas_call(
        _k1_body,
        grid=(M // blk,),
        in_specs=[pl.BlockSpec((blk, 4), lambda i: (i, 0)),
                  pl.BlockSpec((blk, C), lambda i: (i, 0))],
        out_specs=pl.BlockSpec((blk, ROWW), lambda i: (i, 0)),
        out_shape=jax.ShapeDtypeStruct((M, ROWW), _f32),
    )(v_indices, v_features)


def _k2_body(zi_hbm, yi_hbm, xi_hbm, grid_hbm,
             negbuf, zv, yv, xv, linb, valb, sem):
    cid = lax.axis_index("c")
    sid = lax.axis_index("s")

    @pl.when(cid == 0)
    def _memset():
        def fill(i, carry):
            negbuf[pl.ds(i * 16, 16)] = jnp.full((16,), -1, _i32)
            return carry
        lax.fori_loop(0, MEMSET_BUF // 16, fill, 0)
        base = sid * MEMSET_PER_W
        nfull = MEMSET_PER_W // MEMSET_BUF
        tail = MEMSET_PER_W - nfull * MEMSET_BUF

        def fire(i, carry):
            pltpu.async_copy(negbuf, grid_hbm.at[pl.ds(base + i * MEMSET_BUF,
                                                       MEMSET_BUF)], sem)
            return carry
        lax.fori_loop(0, nfull, fire, 0)
        if tail:
            pltpu.async_copy(negbuf.at[pl.ds(0, tail)],
                             grid_hbm.at[pl.ds(base + nfull * MEMSET_BUF, tail)],
                             sem)

        def drain(i, carry):
            pltpu.make_async_copy(
                negbuf, grid_hbm.at[pl.ds(base + i * MEMSET_BUF, MEMSET_BUF)],
                sem).wait()
            return carry
        lax.fori_loop(0, nfull, drain, 0)
        if tail:
            pltpu.make_async_copy(
                negbuf.at[pl.ds(0, tail)],
                grid_hbm.at[pl.ds(base + nfull * MEMSET_BUF, tail)], sem).wait()

    plsc.subcore_barrier()

    @pl.when(cid == 0)
    def _scatter():
        vbase = sid * VOX_PER_W2
        pltpu.sync_copy(zi_hbm.at[pl.ds(vbase, VOX_PER_W2)], zv)
        pltpu.sync_copy(yi_hbm.at[pl.ds(vbase, VOX_PER_W2)], yv)
        pltpu.sync_copy(xi_hbm.at[pl.ds(vbase, VOX_PER_W2)], xv)
        iot = lax.iota(_i32, 16)
        for g in range(VOX_PER_W2 // 16):
            z16 = zv[pl.ds(g * 16, 16)]
            y16 = yv[pl.ds(g * 16, 16)]
            x16 = xv[pl.ds(g * 16, 16)]
            lin = z16 * (YMAX * XMAX) + y16 * XMAX + x16
            val = vbase + g * 16 + iot
            j, col = divmod(g * 16, CHUNK)
            linb[j, pl.ds(col, 16)] = lin
            valb[j, pl.ds(col, 16)] = val
        nscat = VOX_PER_W2 // CHUNK
        for j in range(nscat):
            pltpu.async_copy(valb.at[j], grid_hbm.at[linb.at[j]], sem)
        for j in range(nscat):
            pltpu.make_async_copy(valb.at[j], grid_hbm.at[linb.at[j]],
                                  sem).wait()


def _build_grid(zi, yi, xi):
    mesh = plsc.VectorSubcoreMesh(core_axis_name="c", subcore_axis_name="s",
                                  num_cores=NCORE, num_subcores=NSUB)
    return pl.kernel(
        _k2_body,
        out_type=jax.ShapeDtypeStruct((GRID,), _i32),
        mesh=mesh,
        scratch_types=[
            pltpu.VMEM((MEMSET_BUF,), _i32),
            pltpu.VMEM((VOX_PER_W2,), _i32),
            pltpu.VMEM((VOX_PER_W2,), _i32),
            pltpu.VMEM((VOX_PER_W2,), _i32),
            pltpu.VMEM((VOX_PER_W2 // CHUNK, CHUNK), _i32),
            pltpu.VMEM((VOX_PER_W2 // CHUNK, CHUNK), _i32),
            pltpu.SemaphoreType.DMA,
        ],
    )(zi, yi, xi)


NBUF = 6


def _k3_body(px_hbm, py_hbm, pz_hbm, grid_hbm, raw_hbm, grp_hbm, msk_hbm,
             pxv, pyv, pzv, linbuf, auxbuf, gbuf, *bufs_sems):
    rowbufs = bufs_sems[:NBUF]
    sem_g = bufs_sems[NBUF]
    gsems = bufs_sems[NBUF + 1:2 * NBUF + 1]
    wsems = bufs_sems[2 * NBUF + 1:]
    cid = lax.axis_index("c")
    sid = lax.axis_index("s")
    w = cid * NSUB + sid
    nbase = w * PTS_PER_W
    pbase = nbase * NS

    pltpu.sync_copy(px_hbm.at[pl.ds(nbase, PTS_PER_W)], pxv)
    pltpu.sync_copy(py_hbm.at[pl.ds(nbase, PTS_PER_W)], pyv)
    pltpu.sync_copy(pz_hbm.at[pl.ds(nbase, PTS_PER_W)], pzv)

    def grp(g, carry):
        x16 = pxv[pl.ds(g * 16, 16)]
        y16 = pyv[pl.ds(g * 16, 16)]
        z16 = pzv[pl.ds(g * 16, 16)]
        pxi = ((x16 - XMIN) / VX).astype(_i32)
        pyi = ((y16 - YMIN) / VY).astype(_i32)
        pzi = ((z16 - ZMIN) / VZ).astype(_i32)
        pxi = jnp.minimum(jnp.maximum(pxi, 0), XMAX - 1)
        pyi = jnp.minimum(jnp.maximum(pyi, 0), YMAX - 1)
        pzi = jnp.minimum(jnp.maximum(pzi, 0), ZMAX - 1)
        for k, (dz, dy, dx) in enumerate(_OFFS):
            nz = pzi + dz
            ny = pyi + dy
            nx = pxi + dx
            inb = ((nz >= 0) & (nz < ZMAX) & (ny >= 0) & (ny < YMAX)
                   & (nx >= 0) & (nx < XMAX))
            lin = nz * (YMAX * XMAX) + ny * XMAX + nx
            lin = jnp.where(inb, lin, -1)
            linbuf[pl.ds(k * PTS_PER_W + g * 16, 16)] = lin
        return carry
    lax.fori_loop(0, PTS_PER_W // 16, grp, 0)

    iot = lax.iota(_i32, 16)

    def clampf(i, carry):
        l16 = linbuf[pl.ds(i * 16, 16)]
        pad = (i * 16 + iot) * 13
        auxbuf[pl.ds(i * 16, 16)] = jnp.where(l16 >= 0, l16, pad)
        return carry
    lax.fori_loop(0, IDX_PER_W // 16, clampf, 0)

    def fire_g(c, carry):
        pltpu.async_copy(grid_hbm.at[auxbuf.at[pl.ds(c * CHUNK, CHUNK)]],
                         gbuf.at[pl.ds(c * CHUNK, CHUNK)], sem_g)
        return carry
    lax.fori_loop(0, NCHUNK, fire_g, 0)

    def drain_g(c, carry):
        pltpu.make_async_copy(grid_hbm.at[auxbuf.at[pl.ds(c * CHUNK, CHUNK)]],
                              gbuf.at[pl.ds(c * CHUNK, CHUNK)], sem_g).wait()
        return carry
    lax.fori_loop(0, NCHUNK, drain_g, 0)

    def fpass(i, carry):
        g16 = gbuf[pl.ds(i * 16, 16)]
        l16 = linbuf[pl.ds(i * 16, 16)]
        valid = (g16 >= 0) & (l16 >= 0)
        pad = (i * 16 + iot) & (M - 1)
        auxbuf[pl.ds(i * 16, 16)] = jnp.where(valid, g16, pad)
        gbuf[pl.ds(i * 16, 16)] = jnp.where(valid, 1, 0)
        return carry
    lax.fori_loop(0, IDX_PER_W // 16, fpass, 0)
    pltpu.sync_copy(gbuf, msk_hbm.at[pl.ds(w * IDX_PER_W, IDX_PER_W)])

    def ring(o, carry):
        for b in range(NBUF):
            c = o * NBUF + b

            @pl.when(o > 0)
            def _wait_wb(b=b, c=c):
                pltpu.make_async_copy(
                    rowbufs[b],
                    grp_hbm.at[pl.ds(pbase + (c - NBUF) * CHUNK, CHUNK)],
                    wsems[b]).wait()
            pltpu.async_copy(raw_hbm.at[auxbuf.at[pl.ds(c * CHUNK, CHUNK)]],
                             rowbufs[b], gsems[b])
        for b in range(NBUF):
            c = o * NBUF + b
            pltpu.make_async_copy(
                raw_hbm.at[auxbuf.at[pl.ds(c * CHUNK, CHUNK)]],
                rowbufs[b], gsems[b]).wait()
            pltpu.async_copy(rowbufs[b],
                             grp_hbm.at[pl.ds(pbase + c * CHUNK, CHUNK)],
                             wsems[b])
        return carry
    lax.fori_loop(0, NCHUNK // NBUF, ring, 0)
    for b in range(NBUF):
        c = (NCHUNK - NBUF) + b
        pltpu.make_async_copy(rowbufs[b],
                              grp_hbm.at[pl.ds(pbase + c * CHUNK, CHUNK)],
                              wsems[b]).wait()


def _gather_grouped(px, py, pz, grid, raw):
    mesh = plsc.VectorSubcoreMesh(core_axis_name="c", subcore_axis_name="s",
                                  num_cores=NCORE, num_subcores=NSUB)
    return pl.kernel(
        _k3_body,
        out_type=[jax.ShapeDtypeStruct((PAIRS, ROWW), _f32),
                  jax.ShapeDtypeStruct((PAIRS,), _i32)],
        mesh=mesh,
        compiler_params=pltpu.CompilerParams(use_tc_tiling_on_sc=False),
        scratch_types=[
            pltpu.VMEM((PTS_PER_W,), _f32),
            pltpu.VMEM((PTS_PER_W,), _f32),
            pltpu.VMEM((PTS_PER_W,), _f32),
            pltpu.VMEM((IDX_PER_W,), _i32),
            pltpu.VMEM((IDX_PER_W,), _i32),
            pltpu.VMEM((IDX_PER_W,), _i32),
        ] + [pltpu.VMEM((CHUNK, ROWW), _f32) for _ in range(NBUF)]
          + [pltpu.SemaphoreType.DMA for _ in range(2 * NBUF + 1)],
    )(px, py, pz, grid, raw)


_K4_P = 256
_K4_STEPS = N // _K4_P


_K4_J = PTS_PER_W // _K4_P


def _k4_body(grp_ref, msk_ref, r0_ref, pc_ref, wf_ref, w4_ref,
             hmax_ref, hmin_ref, sums_ref, acc_ref):
    w = pl.program_id(0)
    j = pl.program_id(1)
    raw3 = grp_ref[...][0]
    msk3d = msk_ref[...][0] > 0
    c3 = pc_ref[...][:, 1:4]
    wf = wf_ref[...]
    rel = raw3[:, :, 0:3] - c3[None, :, :]
    dist = jnp.sum(rel * rel, axis=2, keepdims=True)
    feats = jnp.concatenate([rel, dist, raw3[:, :, 4:4 + C]], axis=2)
    flat = feats.reshape(NS * _K4_P, C + 4)
    h3 = jnp.dot(flat, wf[:C + 4], preferred_element_type=_f32).reshape(
        NS, _K4_P, C)
    r0 = r0_ref[...]
    rel0 = r0[:, 0:3] - c3
    dist0 = jnp.sum(rel0 * rel0, axis=1, keepdims=True)
    feats0 = jnp.concatenate(
        [rel0, dist0, jnp.broadcast_to(r0[:, 4:4 + C], (_K4_P, C))], axis=1)
    h0 = jnp.dot(feats0, wf[:C + 4], preferred_element_type=_f32)
    h3 = jnp.where(msk3d, h3, h0[None, :, :])
    hmax_ref[...] = jnp.max(h3, axis=0)
    hmin_ref[...] = jnp.min(h3, axis=0)
    h2 = h3.reshape(NS * _K4_P, C)
    stat = jnp.concatenate([jnp.sum(h2, axis=0)[None, :],
                            jnp.sum(h2 * h2, axis=0)[None, :]], axis=0)

    @pl.when((w == 0) & (j == 0))
    def _init():
        acc_ref[...] = jnp.zeros((2, C), _f32)

    acc_ref[...] += stat

    @pl.when((w == NW - 1) & (j == _K4_J - 1))
    def _out():
        sums_ref[...] = acc_ref[...]


def _dense_stage(grp4, msk4, raw0, p_coords, wfull, w4row):
    return pl.pallas_call(
        _k4_body,
        grid=(NW, _K4_J),
        in_specs=[pl.BlockSpec((1, NS, _K4_P, ROWW), lambda w, j: (w, 0, j, 0)),
                  pl.BlockSpec((1, NS, _K4_P, 1), lambda w, j: (w, 0, j, 0)),
                  pl.BlockSpec((1, ROWW), lambda w, j: (0, 0)),
                  pl.BlockSpec((_K4_P, 4), lambda w, j: (w * _K4_J + j, 0)),
                  pl.BlockSpec((ROWW, C), lambda w, j: (0, 0)),
                  pl.BlockSpec((1, C), lambda w, j: (0, 0))],
        out_specs=[pl.BlockSpec((_K4_P, C), lambda w, j: (w * _K4_J + j, 0)),
                   pl.BlockSpec((_K4_P, C), lambda w, j: (w * _K4_J + j, 0)),
                   pl.BlockSpec((2, C), lambda w, j: (0, 0))],
        out_shape=[jax.ShapeDtypeStruct((N, C), _f32),
                   jax.ShapeDtypeStruct((N, C), _f32),
                   jax.ShapeDtypeStruct((2, C), _f32)],
        scratch_shapes=[pltpu.VMEM((2, C), _f32)],
    )(grp4, msk4, raw0, p_coords, wfull, w4row)


_K5_B = 1024


def _k5_body(sums_ref, hmax_ref, hmin_ref, pf_ref, g_ref, b_ref, out_ref):
    inv = _f32(1.0 / PAIRS)
    mu = sums_ref[0:1, :] * inv
    ex2 = sums_ref[1:2, :] * inv
    var = ex2 - mu * mu
    s = g_ref[...] / jnp.sqrt(var + 1e-5)
    b = b_ref[...] - mu * s
    cand = jnp.where(s >= 0, s * hmax_ref[...] + b, s * hmin_ref[...] + b)
    out_ref[...] = pf_ref[...] + jnp.maximum(cand, 0.0)


def _finalize(sums, hmax, hmin, p_features, gamma2, beta2):
    return pl.pallas_call(
        _k5_body,
        grid=(N // _K5_B,),
        in_specs=[pl.BlockSpec((2, C), lambda i: (0, 0)),
                  pl.BlockSpec((_K5_B, C), lambda i: (i, 0)),
                  pl.BlockSpec((_K5_B, C), lambda i: (i, 0)),
                  pl.BlockSpec((_K5_B, C), lambda i: (i, 0)),
                  pl.BlockSpec((1, C), lambda i: (0, 0)),
                  pl.BlockSpec((1, C), lambda i: (0, 0))],
        out_specs=pl.BlockSpec((_K5_B, C), lambda i: (i, 0)),
        out_shape=jax.ShapeDtypeStruct((N, C), _f32),
    )(sums, hmax, hmin, p_features, gamma2, beta2)


def kernel(v_features, v_indices, p_coords, p_features, W, bn_gamma, bn_beta):
    zi = v_indices[:, 1]
    yi = v_indices[:, 2]
    xi = v_indices[:, 3]
    px = p_coords[:, 1]
    py = p_coords[:, 2]
    pz = p_coords[:, 3]

    raw = _build_raw(v_indices, v_features)
    grid = _build_grid(zi, yi, xi)
    grp, msk = _gather_grouped(px, py, pz, grid, raw)
    grp4 = grp.reshape(NW, NS, PTS_PER_W, ROWW)
    msk4 = msk.reshape(NW, NS, PTS_PER_W, 1)
    raw0 = lax.slice(raw, (0, 0), (1, ROWW))
    wfull = jnp.concatenate([W.T, jnp.zeros((ROWW - 4 - C, C), _f32)], axis=0)
    w4row = W.T[3:4]
    hmax, hmin, sums = _dense_stage(grp4, msk4, raw0, p_coords, wfull, w4row)
    out = _finalize(sums, hmax, hmin, p_features,
                    bn_gamma.reshape(1, C), bn_beta.reshape(1, C))
    return out


# speedup vs baseline: 1.0813x; 1.0253x over previous
import functools

import jax
import jax.numpy as jnp
from jax import lax
from jax.experimental import pallas as pl
from jax.experimental.pallas import tpu as pltpu
from jax.experimental.pallas import tpu_sc as plsc

ZMAX, YMAX, XMAX = 40, 400, 352
GRID = ZMAX * YMAX * XMAX
VX, VY, VZ = 0.2, 0.2, 0.1
XMIN, YMIN, ZMIN = 0.0, -40.0, -2.0
M = 16384
N = 16384
C = 64
NS = 27
ROWW = 72
PAIRS = N * NS

NCORE, NSUB = 2, 16
NW = NCORE * NSUB

MEMSET_PER_W = GRID // NSUB
MEMSET_BUF = 16384
VOX_PER_W2 = M // NSUB

PTS_PER_W = N // NW
IDX_PER_W = PTS_PER_W * NS
CHUNK = 128
NCHUNK = IDX_PER_W // CHUNK

_OFFS = [(a, b, c) for a in (-1, 0, 1) for b in (-1, 0, 1) for c in (-1, 0, 1)]

_f32 = jnp.float32
_i32 = jnp.int32


def _k1_body(vi_ref, vf_ref, raw_ref):
    vi = vi_ref[...]
    b = vi.shape[0]
    xf = (vi[:, 3:4].astype(_f32) + 0.5) * VX + XMIN
    yf = (vi[:, 2:3].astype(_f32) + 0.5) * VY + YMIN
    zf = (vi[:, 1:2].astype(_f32) + 0.5) * VZ + ZMIN
    qf = xf * xf + yf * yf + zf * zf
    raw_ref[...] = jnp.concatenate(
        [xf, yf, zf, qf, vf_ref[...],
         jnp.zeros((b, ROWW - 4 - C), _f32)], axis=1)


def _build_raw(v_indices, v_features):
    blk = 1024
    return pl.pallas_call(
        _k1_body,
        grid=(M // blk,),
        in_specs=[pl.BlockSpec((blk, 4), lambda i: (i, 0)),
                  pl.BlockSpec((blk, C), lambda i: (i, 0))],
        out_specs=pl.BlockSpec((blk, ROWW), lambda i: (i, 0)),
        out_shape=jax.ShapeDtypeStruct((M, ROWW), _f32),
    )(v_indices, v_features)


def _k2_body(zi_hbm, yi_hbm, xi_hbm, grid_hbm,
             negbuf, zv, yv, xv, linb, valb, sem):
    cid = lax.axis_index("c")
    sid = lax.axis_index("s")

    @pl.when(cid == 0)
    def _memset():
        def fill(i, carry):
            negbuf[pl.ds(i * 16, 16)] = jnp.full((16,), -1, _i32)
            return carry
        lax.fori_loop(0, MEMSET_BUF // 16, fill, 0)
        base = sid * MEMSET_PER_W
        nfull = MEMSET_PER_W // MEMSET_BUF
        tail = MEMSET_PER_W - nfull * MEMSET_BUF

        def fire(i, carry):
            pltpu.async_copy(negbuf, grid_hbm.at[pl.ds(base + i * MEMSET_BUF,
                                                       MEMSET_BUF)], sem)
            return carry
        lax.fori_loop(0, nfull, fire, 0)
        if tail:
            pltpu.async_copy(negbuf.at[pl.ds(0, tail)],
                             grid_hbm.at[pl.ds(base + nfull * MEMSET_BUF, tail)],
                             sem)

        def drain(i, carry):
            pltpu.make_async_copy(
                negbuf, grid_hbm.at[pl.ds(base + i * MEMSET_BUF, MEMSET_BUF)],
                sem).wait()
            return carry
        lax.fori_loop(0, nfull, drain, 0)
        if tail:
            pltpu.make_async_copy(
                negbuf.at[pl.ds(0, tail)],
                grid_hbm.at[pl.ds(base + nfull * MEMSET_BUF, tail)], sem).wait()

    plsc.subcore_barrier()

    @pl.when(cid == 0)
    def _scatter():
        vbase = sid * VOX_PER_W2
        pltpu.sync_copy(zi_hbm.at[pl.ds(vbase, VOX_PER_W2)], zv)
        pltpu.sync_copy(yi_hbm.at[pl.ds(vbase, VOX_PER_W2)], yv)
        pltpu.sync_copy(xi_hbm.at[pl.ds(vbase, VOX_PER_W2)], xv)
        iot = lax.iota(_i32, 16)
        for g in range(VOX_PER_W2 // 16):
            z16 = zv[pl.ds(g * 16, 16)]
            y16 = yv[pl.ds(g * 16, 16)]
            x16 = xv[pl.ds(g * 16, 16)]
            lin = z16 * (YMAX * XMAX) + y16 * XMAX + x16
            val = vbase + g * 16 + iot
            j, col = divmod(g * 16, CHUNK)
            linb[j, pl.ds(col, 16)] = lin
            valb[j, pl.ds(col, 16)] = val
        nscat = VOX_PER_W2 // CHUNK
        for j in range(nscat):
            pltpu.async_copy(valb.at[j], grid_hbm.at[linb.at[j]], sem)
        for j in range(nscat):
            pltpu.make_async_copy(valb.at[j], grid_hbm.at[linb.at[j]],
                                  sem).wait()


def _build_grid(zi, yi, xi):
    mesh = plsc.VectorSubcoreMesh(core_axis_name="c", subcore_axis_name="s",
                                  num_cores=NCORE, num_subcores=NSUB)
    return pl.kernel(
        _k2_body,
        out_type=jax.ShapeDtypeStruct((GRID,), _i32),
        mesh=mesh,
        scratch_types=[
            pltpu.VMEM((MEMSET_BUF,), _i32),
            pltpu.VMEM((VOX_PER_W2,), _i32),
            pltpu.VMEM((VOX_PER_W2,), _i32),
            pltpu.VMEM((VOX_PER_W2,), _i32),
            pltpu.VMEM((VOX_PER_W2 // CHUNK, CHUNK), _i32),
            pltpu.VMEM((VOX_PER_W2 // CHUNK, CHUNK), _i32),
            pltpu.SemaphoreType.DMA,
        ],
    )(zi, yi, xi)


NBUF = 6


def _k3_body(px_hbm, py_hbm, pz_hbm, grid_hbm, raw_hbm, grp_hbm, msk_hbm,
             pxv, pyv, pzv, linbuf, auxbuf, gbuf, *bufs_sems):
    rowbufs = bufs_sems[:NBUF]
    sem_g = bufs_sems[NBUF]
    gsems = bufs_sems[NBUF + 1:2 * NBUF + 1]
    wsems = bufs_sems[2 * NBUF + 1:]
    cid = lax.axis_index("c")
    sid = lax.axis_index("s")
    w = cid * NSUB + sid
    nbase = w * PTS_PER_W
    pbase = nbase * NS

    pltpu.sync_copy(px_hbm.at[pl.ds(nbase, PTS_PER_W)], pxv)
    pltpu.sync_copy(py_hbm.at[pl.ds(nbase, PTS_PER_W)], pyv)
    pltpu.sync_copy(pz_hbm.at[pl.ds(nbase, PTS_PER_W)], pzv)

    def grp(g, carry):
        x16 = pxv[pl.ds(g * 16, 16)]
        y16 = pyv[pl.ds(g * 16, 16)]
        z16 = pzv[pl.ds(g * 16, 16)]
        pxi = ((x16 - XMIN) / VX).astype(_i32)
        pyi = ((y16 - YMIN) / VY).astype(_i32)
        pzi = ((z16 - ZMIN) / VZ).astype(_i32)
        pxi = jnp.minimum(jnp.maximum(pxi, 0), XMAX - 1)
        pyi = jnp.minimum(jnp.maximum(pyi, 0), YMAX - 1)
        pzi = jnp.minimum(jnp.maximum(pzi, 0), ZMAX - 1)
        for k, (dz, dy, dx) in enumerate(_OFFS):
            nz = pzi + dz
            ny = pyi + dy
            nx = pxi + dx
            inb = ((nz >= 0) & (nz < ZMAX) & (ny >= 0) & (ny < YMAX)
                   & (nx >= 0) & (nx < XMAX))
            lin = nz * (YMAX * XMAX) + ny * XMAX + nx
            lin = jnp.where(inb, lin, -1)
            linbuf[pl.ds(k * PTS_PER_W + g * 16, 16)] = lin
        return carry
    lax.fori_loop(0, PTS_PER_W // 16, grp, 0)

    iot = lax.iota(_i32, 16)

    def clampf(i, carry):
        l16 = linbuf[pl.ds(i * 16, 16)]
        pad = (i * 16 + iot) * 13
        auxbuf[pl.ds(i * 16, 16)] = jnp.where(l16 >= 0, l16, pad)
        return carry
    lax.fori_loop(0, IDX_PER_W // 16, clampf, 0)

    def fire_g(c, carry):
        pltpu.async_copy(grid_hbm.at[auxbuf.at[pl.ds(c * CHUNK, CHUNK)]],
                         gbuf.at[pl.ds(c * CHUNK, CHUNK)], sem_g)
        return carry
    lax.fori_loop(0, NCHUNK, fire_g, 0)

    def drain_g(c, carry):
        pltpu.make_async_copy(grid_hbm.at[auxbuf.at[pl.ds(c * CHUNK, CHUNK)]],
                              gbuf.at[pl.ds(c * CHUNK, CHUNK)], sem_g).wait()
        return carry
    lax.fori_loop(0, NCHUNK, drain_g, 0)

    def fpass(i, carry):
        g16 = gbuf[pl.ds(i * 16, 16)]
        l16 = linbuf[pl.ds(i * 16, 16)]
        valid = (g16 >= 0) & (l16 >= 0)
        pad = (i * 16 + iot) & (M - 1)
        auxbuf[pl.ds(i * 16, 16)] = jnp.where(valid, g16, pad)
        gbuf[pl.ds(i * 16, 16)] = jnp.where(valid, 1, 0)
        return carry
    lax.fori_loop(0, IDX_PER_W // 16, fpass, 0)
    pltpu.sync_copy(gbuf, msk_hbm.at[pl.ds(w * IDX_PER_W, IDX_PER_W)])

    def ring(o, carry):
        for b in range(NBUF):
            c = o * NBUF + b

            @pl.when(o > 0)
            def _wait_wb(b=b, c=c):
                pltpu.make_async_copy(
                    rowbufs[b],
                    grp_hbm.at[pl.ds(pbase + (c - NBUF) * CHUNK, CHUNK)],
                    wsems[b]).wait()
            pltpu.async_copy(raw_hbm.at[auxbuf.at[pl.ds(c * CHUNK, CHUNK)]],
                             rowbufs[b], gsems[b])
        for b in range(NBUF):
            c = o * NBUF + b
            pltpu.make_async_copy(
                raw_hbm.at[auxbuf.at[pl.ds(c * CHUNK, CHUNK)]],
                rowbufs[b], gsems[b]).wait()
            pltpu.async_copy(rowbufs[b],
                             grp_hbm.at[pl.ds(pbase + c * CHUNK, CHUNK)],
                             wsems[b])
        return carry
    lax.fori_loop(0, NCHUNK // NBUF, ring, 0)
    for b in range(NBUF):
        c = (NCHUNK - NBUF) + b
        pltpu.make_async_copy(rowbufs[b],
                              grp_hbm.at[pl.ds(pbase + c * CHUNK, CHUNK)],
                              wsems[b]).wait()


def _gather_grouped(px, py, pz, grid, raw):
    mesh = plsc.VectorSubcoreMesh(core_axis_name="c", subcore_axis_name="s",
                                  num_cores=NCORE, num_subcores=NSUB)
    return pl.kernel(
        _k3_body,
        out_type=[jax.ShapeDtypeStruct((PAIRS, ROWW), _f32),
                  jax.ShapeDtypeStruct((PAIRS,), _i32)],
        mesh=mesh,
        compiler_params=pltpu.CompilerParams(use_tc_tiling_on_sc=False),
        scratch_types=[
            pltpu.VMEM((PTS_PER_W,), _f32),
            pltpu.VMEM((PTS_PER_W,), _f32),
            pltpu.VMEM((PTS_PER_W,), _f32),
            pltpu.VMEM((IDX_PER_W,), _i32),
            pltpu.VMEM((IDX_PER_W,), _i32),
            pltpu.VMEM((IDX_PER_W,), _i32),
        ] + [pltpu.VMEM((CHUNK, ROWW), _f32) for _ in range(NBUF)]
          + [pltpu.SemaphoreType.DMA for _ in range(2 * NBUF + 1)],
    )(px, py, pz, grid, raw)


_K4_P = 512
_K4_STEPS = N // _K4_P


_K4_J = PTS_PER_W // _K4_P


def _k4_body(grp_ref, msk_ref, r0_ref, pc_ref, wf_ref, w4_ref,
             hmax_ref, hmin_ref, sums_ref, acc_ref):
    w = pl.program_id(0)
    j = pl.program_id(1)
    raw3 = grp_ref[...][0]
    msk3d = msk_ref[...][0] > 0
    c3 = pc_ref[...][:, 1:4]
    wf = wf_ref[...]
    rel = raw3[:, :, 0:3] - c3[None, :, :]
    dist = jnp.sum(rel * rel, axis=2, keepdims=True)
    feats = jnp.concatenate([rel, dist, raw3[:, :, 4:4 + C]], axis=2)
    flat = feats.reshape(NS * _K4_P, C + 4)
    h3 = jnp.dot(flat, wf[:C + 4], preferred_element_type=_f32).reshape(
        NS, _K4_P, C)
    r0 = r0_ref[...]
    rel0 = r0[:, 0:3] - c3
    dist0 = jnp.sum(rel0 * rel0, axis=1, keepdims=True)
    feats0 = jnp.concatenate(
        [rel0, dist0, jnp.broadcast_to(r0[:, 4:4 + C], (_K4_P, C))], axis=1)
    h0 = jnp.dot(feats0, wf[:C + 4], preferred_element_type=_f32)
    h3 = jnp.where(msk3d, h3, h0[None, :, :])
    hmax_ref[...] = jnp.max(h3, axis=0)
    hmin_ref[...] = jnp.min(h3, axis=0)
    h2 = h3.reshape(NS * _K4_P, C)
    stat = jnp.concatenate([jnp.sum(h2, axis=0)[None, :],
                            jnp.sum(h2 * h2, axis=0)[None, :]], axis=0)

    @pl.when((w == 0) & (j == 0))
    def _init():
        acc_ref[...] = jnp.zeros((2, C), _f32)

    acc_ref[...] += stat

    @pl.when((w == NW - 1) & (j == _K4_J - 1))
    def _out():
        sums_ref[...] = acc_ref[...]


def _dense_stage(grp4, msk4, raw0, p_coords, wfull, w4row):
    return pl.pallas_call(
        _k4_body,
        grid=(NW, _K4_J),
        in_specs=[pl.BlockSpec((1, NS, _K4_P, ROWW), lambda w, j: (w, 0, j, 0)),
                  pl.BlockSpec((1, NS, _K4_P, 1), lambda w, j: (w, 0, j, 0)),
                  pl.BlockSpec((1, ROWW), lambda w, j: (0, 0)),
                  pl.BlockSpec((_K4_P, 4), lambda w, j: (w * _K4_J + j, 0)),
                  pl.BlockSpec((ROWW, C), lambda w, j: (0, 0)),
                  pl.BlockSpec((1, C), lambda w, j: (0, 0))],
        out_specs=[pl.BlockSpec((_K4_P, C), lambda w, j: (w * _K4_J + j, 0)),
                   pl.BlockSpec((_K4_P, C), lambda w, j: (w * _K4_J + j, 0)),
                   pl.BlockSpec((2, C), lambda w, j: (0, 0))],
        out_shape=[jax.ShapeDtypeStruct((N, C), _f32),
                   jax.ShapeDtypeStruct((N, C), _f32),
                   jax.ShapeDtypeStruct((2, C), _f32)],
        scratch_shapes=[pltpu.VMEM((2, C), _f32)],
    )(grp4, msk4, raw0, p_coords, wfull, w4row)


_K5_B = 1024


def _k5_body(sums_ref, hmax_ref, hmin_ref, pf_ref, g_ref, b_ref, out_ref):
    inv = _f32(1.0 / PAIRS)
    mu = sums_ref[0:1, :] * inv
    ex2 = sums_ref[1:2, :] * inv
    var = ex2 - mu * mu
    s = g_ref[...] / jnp.sqrt(var + 1e-5)
    b = b_ref[...] - mu * s
    cand = jnp.where(s >= 0, s * hmax_ref[...] + b, s * hmin_ref[...] + b)
    out_ref[...] = pf_ref[...] + jnp.maximum(cand, 0.0)


def _finalize(sums, hmax, hmin, p_features, gamma2, beta2):
    return pl.pallas_call(
        _k5_body,
        grid=(N // _K5_B,),
        in_specs=[pl.BlockSpec((2, C), lambda i: (0, 0)),
                  pl.BlockSpec((_K5_B, C), lambda i: (i, 0)),
                  pl.BlockSpec((_K5_B, C), lambda i: (i, 0)),
                  pl.BlockSpec((_K5_B, C), lambda i: (i, 0)),
                  pl.BlockSpec((1, C), lambda i: (0, 0)),
                  pl.BlockSpec((1, C), lambda i: (0, 0))],
        out_specs=pl.BlockSpec((_K5_B, C), lambda i: (i, 0)),
        out_shape=jax.ShapeDtypeStruct((N, C), _f32),
    )(sums, hmax, hmin, p_features, gamma2, beta2)


def kernel(v_features, v_indices, p_coords, p_features, W, bn_gamma, bn_beta):
    zi = v_indices[:, 1]
    yi = v_indices[:, 2]
    xi = v_indices[:, 3]
    px = p_coords[:, 1]
    py = p_coords[:, 2]
    pz = p_coords[:, 3]

    raw = _build_raw(v_indices, v_features)
    grid = _build_grid(zi, yi, xi)
    grp, msk = _gather_grouped(px, py, pz, grid, raw)
    grp4 = grp.reshape(NW, NS, PTS_PER_W, ROWW)
    msk4 = msk.reshape(NW, NS, PTS_PER_W, 1)
    raw0 = lax.slice(raw, (0, 0), (1, ROWW))
    wfull = jnp.concatenate([W.T, jnp.zeros((ROWW - 4 - C, C), _f32)], axis=0)
    w4row = W.T[3:4]
    hmax, hmin, sums = _dense_stage(grp4, msk4, raw0, p_coords, wfull, w4row)
    out = _finalize(sums, hmax, hmin, p_features,
                    bn_gamma.reshape(1, C), bn_beta.reshape(1, C))
    return out
